# baseline (speedup 1.0000x reference)
import functools

import jax
import jax.numpy as jnp
from jax import lax
from jax.experimental import pallas as pl
from jax.experimental.pallas import tpu as pltpu
from jax.experimental.pallas import tpu_sc as plsc

N_NODES = 10000
D = 128
NPAD = 10240
NC, NS = 2, 16
NW = NC * NS
CHUNK = 128
RPT = NPAD // NS


def _deg_kernel(n_chunks):
    mesh = plsc.VectorSubcoreMesh(core_axis_name="c", subcore_axis_name="s")

    @functools.partial(
        pl.kernel, mesh=mesh,
        out_type=jax.ShapeDtypeStruct((NC * NPAD, 16), jnp.float32),
        scratch_types=[
            pltpu.VMEM_SHARED((NPAD, 16), jnp.float32),
            pltpu.VMEM((128, 16), jnp.float32),
            pltpu.VMEM((CHUNK,), jnp.int32),
            pltpu.VMEM((RPT, 16), jnp.float32),
        ],
    )
    def k(dst3, out, deg_sp, ones_v, idx_v, stage_v):
        c = lax.axis_index("c")
        s = lax.axis_index("s")
        wid = s * NC + c

        def fill(i, _):
            ones_v[i] = jnp.ones((16,), jnp.float32)
            return 0
        lax.fori_loop(0, 128, fill, 0)

        def zfill(i, _):
            stage_v[i] = jnp.zeros((16,), jnp.float32)
            return 0
        lax.fori_loop(0, RPT, zfill, 0)
        pltpu.sync_copy(stage_v, deg_sp.at[pl.ds(s * RPT, RPT)])
        plsc.subcore_barrier()

        def body(j, _):
            pltpu.sync_copy(dst3.at[wid, j], idx_v)
            pltpu.sync_copy(ones_v, deg_sp.at[idx_v], add=True)
            return 0
        lax.fori_loop(0, n_chunks, body, 0)
        plsc.subcore_barrier()

        pltpu.sync_copy(deg_sp.at[pl.ds(s * RPT, RPT)], stage_v)
        pltpu.sync_copy(stage_v, out.at[pl.ds(c * NPAD + s * RPT, RPT)])

    return k


NBUF = 2
NIDX = 4


def _scatter_kernel(n_chunks):
    assert n_chunks % NBUF == 0
    mesh = plsc.VectorSubcoreMesh(core_axis_name="c", subcore_axis_name="s")

    @functools.partial(
        pl.kernel, mesh=mesh,
        out_type=jax.ShapeDtypeStruct((NC * NPAD, D), jnp.float32),
        scratch_types=[
            pltpu.VMEM_SHARED((NPAD, D), jnp.float32),
            pltpu.VMEM((n_chunks, CHUNK), jnp.int32),
        ] + [pltpu.VMEM((CHUNK,), jnp.int32)] * NIDX
          + [pltpu.VMEM((CHUNK, D), jnp.float32)] * NBUF
          + [pltpu.SemaphoreType.DMA] * (NIDX + 2 * NBUF),
    )
    def k(h, src3, dst3, out, acc_sp, sidx, *rest):
        didx = rest[:NIDX]
        bufs = rest[NIDX:NIDX + NBUF]
        sd = rest[NIDX + NBUF:2 * NIDX + NBUF]
        sg = rest[2 * NIDX + NBUF:2 * NIDX + 2 * NBUF]
        ss = rest[2 * NIDX + 2 * NBUF:]
        c = lax.axis_index("c")
        s = lax.axis_index("s")
        wid = s * NC + c

        def init(t, _):
            base = s * RPT + t * CHUNK
            pltpu.sync_copy(h.at[pl.ds(base, CHUNK)], bufs[0])
            pltpu.sync_copy(bufs[0], acc_sp.at[pl.ds(base, CHUNK)])
            return 0
        lax.fori_loop(0, RPT // CHUNK, init, 0)
        pltpu.sync_copy(src3.at[wid], sidx)
        plsc.subcore_barrier()

        def di_start(j, d):
            pltpu.async_copy(dst3.at[wid, j], didx[d], sd[d])

        def di_wait(d):
            pltpu.make_async_copy(dst3.at[0, 0], didx[d], sd[d]).wait()

        def g_start(j, b):
            pltpu.async_copy(h.at[sidx.at[j]], bufs[b], sg[b])

        def g_wait(b):
            pltpu.make_async_copy(h.at[pl.ds(0, CHUNK)], bufs[b], sg[b]).wait()

        def s_start(d, b):
            pltpu.async_copy(bufs[b], acc_sp.at[didx[d]], ss[b], add=True)

        def s_wait(b):
            pltpu.make_async_copy(bufs[b], acc_sp.at[pl.ds(0, CHUNK)],
                                  ss[b]).wait()

        for j0 in range(min(3, n_chunks)):
            di_start(j0, j0)
        g_start(0, 0)
        if n_chunks > 1:
            g_start(1, 1)

        def group(g, _):
            for u in range(NIDX):
                j = g * NIDX + u
                b = u % NBUF

                @pl.when(j + 3 < n_chunks)
                def _():
                    di_start(j + 3, (u + 3) % NIDX)
                g_wait(b)
                di_wait(u)
                s_start(u, b)
                s_wait(b)

                @pl.when(j + 2 < n_chunks)
                def _():
                    g_start(j + 2, b)
            return 0
        lax.fori_loop(0, n_chunks // NIDX, group, 0)
        plsc.subcore_barrier()

        def wb(t, _):
            base = s * RPT + t * CHUNK
            pltpu.sync_copy(acc_sp.at[pl.ds(base, CHUNK)], bufs[0])
            pltpu.sync_copy(bufs[0], out.at[pl.ds(c * NPAD + base, CHUNK)])
            return 0
        lax.fori_loop(0, RPT // CHUNK, wb, 0)

    return k


def _matmul_body(x_ref, w_ref, da_ref, db_ref, o_ref):
    deg = da_ref[:, 0:1] + db_ref[:, 0:1] + 1.0
    dis = lax.rsqrt(deg)
    o_ref[:, :] = jnp.dot(x_ref[:, :], w_ref[:, :],
                          preferred_element_type=jnp.float32) * dis


def _finish_body(p0_ref, p1_ref, h_ref, da_ref, db_ref, b_ref, g_ref,
                 be_ref, o_ref):
    s_tot = p0_ref[:, :] + p1_ref[:, :] - h_ref[:, :]
    deg = da_ref[:, 0:1] + db_ref[:, 0:1] + 1.0
    dis = lax.rsqrt(deg)
    pre = s_tot * dis + b_ref[:, :]
    valid = pre[0:N_NODES, :]
    mean = jnp.mean(valid, axis=0, keepdims=True)
    cen = valid - mean
    var = jnp.mean(cen * cen, axis=0, keepdims=True)
    y = cen * lax.rsqrt(var + 1e-5) * g_ref[:, :] + be_ref[:, :]
    o_ref[:, :] = jnp.maximum(y, 0.0)


def kernel(x, edge_index, W, b, gamma, beta):
    n_edges = edge_index.shape[1]
    n_chunks = -(-n_edges // (NW * CHUNK))
    n_chunks = -(-n_chunks // NIDX) * NIDX
    epad = n_chunks * NW * CHUNK

    ei = edge_index.astype(jnp.int32)
    pad = jnp.full((epad - n_edges,), N_NODES, jnp.int32)
    src3 = jnp.concatenate([ei[0], pad]).reshape(NW, n_chunks, CHUNK)
    dst3 = jnp.concatenate([ei[1], pad]).reshape(NW, n_chunks, CHUNK)
    x_pad = jnp.pad(x, ((0, NPAD - x.shape[0]), (0, 0)))

    deg2 = _deg_kernel(n_chunks)(dst3)
    deg_a, deg_b = deg2[:NPAD], deg2[NPAD:]

    n_blk = 8
    rows = NPAD // n_blk
    h = pl.pallas_call(
        _matmul_body,
        grid=(n_blk,),
        in_specs=[
            pl.BlockSpec((rows, D), lambda i: (i, 0)),
            pl.BlockSpec((D, D), lambda i: (0, 0)),
            pl.BlockSpec((rows, 16), lambda i: (i, 0)),
            pl.BlockSpec((rows, 16), lambda i: (i, 0)),
        ],
        out_specs=pl.BlockSpec((rows, D), lambda i: (i, 0)),
        out_shape=jax.ShapeDtypeStruct((NPAD, D), jnp.float32),
    )(x_pad, W, deg_a, deg_b)

    p = _scatter_kernel(n_chunks)(h, src3, dst3)
    p0, p1 = p[:NPAD], p[NPAD:]

    out = pl.pallas_call(
        _finish_body,
        out_shape=jax.ShapeDtypeStruct((N_NODES, D), jnp.float32),
    )(p0, p1, h, deg_a, deg_b, b.reshape(1, D), gamma.reshape(1, D),
      beta.reshape(1, D))
    return out


# speedup vs baseline: 1.1106x; 1.1106x over previous
import functools

import jax
import jax.numpy as jnp
from jax import lax
from jax.experimental import pallas as pl
from jax.experimental.pallas import tpu as pltpu
from jax.experimental.pallas import tpu_sc as plsc

N_NODES = 10000
D = 128
NPAD = 10240
NC, NS = 2, 16
NW = NC * NS
CHUNK = 128
RPT = NPAD // NS


def _deg_kernel(n_chunks):
    mesh = plsc.VectorSubcoreMesh(core_axis_name="c", subcore_axis_name="s")

    @functools.partial(
        pl.kernel, mesh=mesh,
        out_type=jax.ShapeDtypeStruct((NC * NPAD, 16), jnp.float32),
        scratch_types=[
            pltpu.VMEM_SHARED((NPAD, 16), jnp.float32),
            pltpu.VMEM((128, 16), jnp.float32),
            pltpu.VMEM((CHUNK,), jnp.int32),
            pltpu.VMEM((RPT, 16), jnp.float32),
        ],
    )
    def k(dst3, out, deg_sp, ones_v, idx_v, stage_v):
        c = lax.axis_index("c")
        s = lax.axis_index("s")
        wid = s * NC + c

        def fill(i, _):
            ones_v[i] = jnp.ones((16,), jnp.float32)
            return 0
        lax.fori_loop(0, 128, fill, 0)

        def zfill(i, _):
            stage_v[i] = jnp.zeros((16,), jnp.float32)
            return 0
        lax.fori_loop(0, RPT, zfill, 0)
        pltpu.sync_copy(stage_v, deg_sp.at[pl.ds(s * RPT, RPT)])
        plsc.subcore_barrier()

        def body(j, _):
            pltpu.sync_copy(dst3.at[wid, j], idx_v)
            pltpu.sync_copy(ones_v, deg_sp.at[idx_v], add=True)
            return 0
        lax.fori_loop(0, n_chunks, body, 0)
        plsc.subcore_barrier()

        pltpu.sync_copy(deg_sp.at[pl.ds(s * RPT, RPT)], stage_v)
        pltpu.sync_copy(stage_v, out.at[pl.ds(c * NPAD + s * RPT, RPT)])

    return k


def _scatter_kernel(n_chunks):
    mesh = plsc.VectorSubcoreMesh(core_axis_name="c", subcore_axis_name="s")

    @functools.partial(
        pl.kernel, mesh=mesh,
        out_type=jax.ShapeDtypeStruct((NC * NPAD, D), jnp.float32),
        scratch_types=[
            pltpu.VMEM_SHARED((NPAD, D), jnp.float32),
            pltpu.VMEM((CHUNK,), jnp.int32),
            pltpu.VMEM((CHUNK,), jnp.int32),
            pltpu.VMEM((CHUNK, D), jnp.float32),
        ],
    )
    def k(h, src3, dst3, out, acc_sp, sidx_v, didx_v, buf_v):
        c = lax.axis_index("c")
        s = lax.axis_index("s")
        wid = s * NC + c

        def init(t, _):
            base = s * RPT + t * CHUNK
            pltpu.sync_copy(h.at[pl.ds(base, CHUNK)], buf_v)
            pltpu.sync_copy(buf_v, acc_sp.at[pl.ds(base, CHUNK)])
            return 0
        lax.fori_loop(0, RPT // CHUNK, init, 0)
        plsc.subcore_barrier()

        def body(j, _):
            pltpu.sync_copy(src3.at[wid, j], sidx_v)
            pltpu.sync_copy(dst3.at[wid, j], didx_v)
            pltpu.sync_copy(h.at[sidx_v], buf_v)
            pltpu.sync_copy(buf_v, acc_sp.at[didx_v], add=True)
            return 0
        lax.fori_loop(0, n_chunks, body, 0)
        plsc.subcore_barrier()

        def wb(t, _):
            base = s * RPT + t * CHUNK
            pltpu.sync_copy(acc_sp.at[pl.ds(base, CHUNK)], buf_v)
            pltpu.sync_copy(buf_v, out.at[pl.ds(c * NPAD + base, CHUNK)])
            return 0
        lax.fori_loop(0, RPT // CHUNK, wb, 0)

    return k


def _matmul_body(x_ref, w_ref, da_ref, db_ref, o_ref):
    deg = da_ref[:, 0:1] + db_ref[:, 0:1] + 1.0
    dis = lax.rsqrt(deg)
    o_ref[:, :] = jnp.dot(x_ref[:, :], w_ref[:, :],
                          preferred_element_type=jnp.float32) * dis


def _finish_body(p_ref, h_ref, da_ref, db_ref, b_ref, g_ref, be_ref, o_ref):
    pre = (p_ref[0, 0:N_NODES, :] + p_ref[1, 0:N_NODES, :]
           - h_ref[0:N_NODES, :])
    deg = da_ref[0:N_NODES, 0:1] + db_ref[0:N_NODES, 0:1] + 1.0
    dis = lax.rsqrt(deg)
    pre = pre * dis + b_ref[:, :]
    mean = jnp.mean(pre, axis=0, keepdims=True)
    cen = pre - mean
    var = jnp.mean(cen * cen, axis=0, keepdims=True)
    y = cen * lax.rsqrt(var + 1e-5) * g_ref[:, :] + be_ref[:, :]
    o_ref[:, :] = jnp.maximum(y, 0.0)


def kernel(x, edge_index, W, b, gamma, beta):
    n_edges = edge_index.shape[1]
    n_chunks = -(-n_edges // (NW * CHUNK))
    epad = n_chunks * NW * CHUNK

    ei = edge_index.astype(jnp.int32)
    pad = jnp.full((epad - n_edges,), N_NODES, jnp.int32)
    src_flat = jnp.concatenate([ei[0], pad])
    dst_flat = jnp.concatenate([ei[1], pad])
    src3 = src_flat.reshape(NW, n_chunks, CHUNK)
    dst3 = dst_flat.reshape(NW, n_chunks, CHUNK)
    x_pad = jnp.pad(x, ((0, NPAD - x.shape[0]), (0, 0)))

    deg2 = _deg_kernel(n_chunks)(dst3)
    deg_a, deg_b = deg2[:NPAD], deg2[NPAD:]

    n_blk = 8
    rows = NPAD // n_blk
    h = pl.pallas_call(
        _matmul_body,
        grid=(n_blk,),
        in_specs=[
            pl.BlockSpec((rows, D), lambda i: (i, 0)),
            pl.BlockSpec((D, D), lambda i: (0, 0)),
            pl.BlockSpec((rows, 16), lambda i: (i, 0)),
            pl.BlockSpec((rows, 16), lambda i: (i, 0)),
        ],
        out_specs=pl.BlockSpec((rows, D), lambda i: (i, 0)),
        out_shape=jax.ShapeDtypeStruct((NPAD, D), jnp.float32),
    )(x_pad, W, deg_a, deg_b)

    p = _scatter_kernel(n_chunks)(h, src3, dst3).reshape(NC, NPAD, D)

    out = pl.pallas_call(
        _finish_body,
        out_shape=jax.ShapeDtypeStruct((N_NODES, D), jnp.float32),
    )(p, h, deg_a, deg_b, b.reshape(1, D), gamma.reshape(1, D),
      beta.reshape(1, D))
    return out
